# baseline (speedup 1.0000x reference)
import jax
import jax.numpy as jnp
from jax import lax
from jax.experimental import pallas as pl
from jax.experimental.pallas import tpu as pltpu

B, D, T = 32, 64, 576
K = 1024
N = B * T
EPS = 1e-12
BB = 16
W = BB * T


def _vq_kernel(x_ref, emb_ref, embu_ref, quant_ref, idx_ref, loss_ref, cat_ref):
    g = pl.program_id(0)

    @pl.when(g == 0)
    def _init():
        loss_ref[...] = jnp.zeros((1, 1), jnp.float32)
        cat_ref[:, :D] = emb_ref[...].astype(jnp.bfloat16)
        cat_ref[:, D:] = embu_ref[...].astype(jnp.bfloat16)

    x = jnp.concatenate([x_ref[i] for i in range(BB)], axis=1)
    emb = emb_ref[...]

    norm = jnp.sqrt(jnp.sum(x * x, axis=0, keepdims=True))
    xn = x / jnp.maximum(norm, EPS)

    dist = lax.dot_general(
        emb, xn, (((1,), (0,)), ((), ())),
        preferred_element_type=jnp.float32,
    )

    maxval = jnp.max(dist, axis=0, keepdims=True)
    iota_f = lax.broadcasted_iota(jnp.int32, (K, W), 0).astype(jnp.float32)
    idxf = jnp.min(jnp.where(dist >= maxval, iota_f, float(K)), axis=0)

    onehot = (iota_f == idxf[None, :]).astype(jnp.bfloat16)
    combo = lax.dot_general(
        cat_ref[...], onehot, (((0,), (0,)), ((), ())),
        preferred_element_type=jnp.float32,
    )

    idx = idxf.astype(jnp.int32)
    for i in range(BB):
        idx_ref[i, 0] = idx[i * T:(i + 1) * T]
        quant_ref[i] = combo[:D, i * T:(i + 1) * T]
    diff = x - combo[D:]
    loss_ref[...] += jnp.sum(diff * diff).reshape(1, 1)


def kernel(inputs, embedding, embedding_unnormalized):
    quant, idx3, loss_sum = pl.pallas_call(
        _vq_kernel,
        grid=(B // BB,),
        in_specs=[
            pl.BlockSpec((BB, D, T), lambda g: (g, 0, 0)),
            pl.BlockSpec((K, D), lambda g: (0, 0)),
            pl.BlockSpec((K, D), lambda g: (0, 0)),
        ],
        out_specs=[
            pl.BlockSpec((BB, D, T), lambda g: (g, 0, 0)),
            pl.BlockSpec((BB, 1, T), lambda g: (g, 0, 0)),
            pl.BlockSpec((1, 1), lambda g: (0, 0)),
        ],
        out_shape=[
            jax.ShapeDtypeStruct((B, D, T), jnp.float32),
            jax.ShapeDtypeStruct((B, 1, T), jnp.int32),
            jax.ShapeDtypeStruct((1, 1), jnp.float32),
        ],
        scratch_shapes=[pltpu.VMEM((K, 2 * D), jnp.bfloat16)],
    )(inputs, embedding, embedding_unnormalized)

    loss = loss_sum[0, 0] / jnp.float32(N * D)
    encoding_indices = idx3.reshape(N)
    return (quant, loss, loss, encoding_indices)


# speedup vs baseline: 1.0729x; 1.0729x over previous
import jax
import jax.numpy as jnp
from jax import lax
from jax.experimental import pallas as pl
from jax.experimental.pallas import tpu as pltpu

B, D, T = 32, 64, 576
K = 1024
N = B * T
EPS = 1e-12
BB = 8
W = BB * T
C = 2 * D + 8
INV_ND = 1.0 / float(N * D)


def _vq_kernel(x_ref, emb_ref, embu_ref, aux_ref, quant_ref, idx_ref, loss_ref,
               cat_ref):
    g = pl.program_id(0)

    @pl.when(g == 0)
    def _init():
        loss_ref[...] = jnp.zeros((1, 1), jnp.float32)
        cat_ref[:, :D] = emb_ref[...].astype(jnp.bfloat16)
        cat_ref[:, D:2 * D] = embu_ref[...].astype(jnp.bfloat16)
        cat_ref[:, 2 * D:] = aux_ref[...]

    x = jnp.concatenate([x_ref[i] for i in range(BB)], axis=1)
    emb = emb_ref[...]

    norm = jnp.sqrt(jnp.sum(x * x, axis=0, keepdims=True))
    xn = x / jnp.maximum(norm, EPS)

    dist = lax.dot_general(
        emb, xn, (((1,), (0,)), ((), ())),
        preferred_element_type=jnp.float32,
    )

    maxval = jnp.max(dist, axis=0, keepdims=True)
    mask = (dist >= maxval).astype(jnp.bfloat16)

    combo = lax.dot_general(
        cat_ref[...], mask, (((0,), (0,)), ((), ())),
        preferred_element_type=jnp.float32,
    )
    idxf = combo[2 * D] * 32.0 + combo[2 * D + 1]
    cnt = combo[2 * D + 2]

    idx = idxf.astype(jnp.int32)
    for i in range(BB):
        idx_ref[i, 0] = idx[i * T:(i + 1) * T]
        quant_ref[i] = combo[:D, i * T:(i + 1) * T]
    diff = x - combo[D:2 * D]
    loss_ref[...] += (jnp.sum(diff * diff) * INV_ND).reshape(1, 1)

    tie = jnp.max(cnt) > 1.5

    @pl.when(tie)
    def _exact():
        iota_f = lax.broadcasted_iota(jnp.int32, (K, W), 0).astype(jnp.float32)
        idxe = jnp.min(jnp.where(dist >= maxval, iota_f, float(K)), axis=0)
        idxi = idxe.astype(jnp.int32)
        for i in range(BB):
            idx_ref[i, 0] = idxi[i * T:(i + 1) * T]


def kernel(inputs, embedding, embedding_unnormalized):
    k_iota = jnp.arange(K, dtype=jnp.int32)
    aux = jnp.stack(
        [(k_iota >> 5).astype(jnp.bfloat16),
         (k_iota & 31).astype(jnp.bfloat16),
         jnp.ones((K,), jnp.bfloat16)]
        + [jnp.zeros((K,), jnp.bfloat16)] * 5,
        axis=1,
    )

    quant, idx3, loss_sum = pl.pallas_call(
        _vq_kernel,
        grid=(B // BB,),
        in_specs=[
            pl.BlockSpec((BB, D, T), lambda g: (g, 0, 0)),
            pl.BlockSpec((K, D), lambda g: (0, 0)),
            pl.BlockSpec((K, D), lambda g: (0, 0)),
            pl.BlockSpec((K, 8), lambda g: (0, 0)),
        ],
        out_specs=[
            pl.BlockSpec((BB, D, T), lambda g: (g, 0, 0)),
            pl.BlockSpec((BB, 1, T), lambda g: (g, 0, 0)),
            pl.BlockSpec((1, 1), lambda g: (0, 0)),
        ],
        out_shape=[
            jax.ShapeDtypeStruct((B, D, T), jnp.float32),
            jax.ShapeDtypeStruct((B, 1, T), jnp.int32),
            jax.ShapeDtypeStruct((1, 1), jnp.float32),
        ],
        scratch_shapes=[pltpu.VMEM((K, C), jnp.bfloat16)],
    )(inputs, embedding, embedding_unnormalized, aux)

    loss = loss_sum.reshape(())
    encoding_indices = idx3.reshape(N)
    return (quant, loss, loss, encoding_indices)
